# baseline (speedup 1.0000x reference)
import functools

import jax
import jax.numpy as jnp
import numpy as np
from jax import lax
from jax.experimental import pallas as pl
from jax.experimental.pallas import tpu as pltpu
from jax.experimental.pallas import tpu_sc as plsc

G, D, P = 20000, 128, 16
H = 128
PCA = 512
B = 16384
N = 32768

HP = H * P
NC, NS, L = 2, 16, 16
B2 = B // NC
SEG = 256
NPASS = B // SEG
WCAP = 2048
GROUP = 16
ROWS_TILE = SEG // NS
NBOUND = 80


def _table_transform_body(x_ref, w_ref, b_ref, th_ref, m_ref):
    x = x_ref[...]
    tg = x.shape[0]
    x2 = x.reshape(tg * P, D)
    y = jnp.dot(x2, w_ref[...], preferred_element_type=jnp.float32)
    y = y + b_ref[...]
    y = jnp.where(y >= 0, y, 0.01 * y)
    th_ref[...] = y.reshape(tg, P, H)
    s = jnp.sum(x, axis=2)
    m = (s != 0).astype(jnp.float32)
    m_ref[...] = jnp.concatenate(
        [m, jnp.zeros((tg, H - P), jnp.float32)], axis=1)


def _table_transform(tt, W1, b1r):
    TG = 400
    grid = G // TG
    return pl.pallas_call(
        _table_transform_body,
        grid=(grid,),
        in_specs=[
            pl.BlockSpec((TG, P, D), lambda i: (i, 0, 0)),
            pl.BlockSpec((D, H), lambda i: (0, 0)),
            pl.BlockSpec((1, H), lambda i: (0, 0)),
        ],
        out_specs=[
            pl.BlockSpec((TG, P, H), lambda i: (i, 0, 0)),
            pl.BlockSpec((TG, H), lambda i: (i, 0)),
        ],
        out_shape=[
            jax.ShapeDtypeStruct((G, P, H), jnp.float32),
            jax.ShapeDtypeStruct((G, H), jnp.float32),
        ],
    )(tt, W1, b1r)


def _seg_body(th_hbm, mg_hbm, lg_hbm, lc_hbm, bnd_hbm, agg_hbm, magg_hbm,
              lgb, lcb, gidx, lcx, gbuf0, gbuf1, mbuf0, mbuf1, zbuf, zmb,
              bnd_v, acc, macc, semv0, semv1, semm0, semm1):
    c = lax.axis_index("c")
    s = lax.axis_index("s")

    pltpu.sync_copy(bnd_hbm, bnd_v)

    def _zb(k, carry):
        zbuf[k // (P * 8), (k // 8) % P, pl.ds((k % 8) * L, L)] = (
            jnp.zeros((L,), jnp.float32))
        return carry
    lax.fori_loop(0, 4 * P * 8, _zb, 0)

    def _zm(k, carry):
        zmb[k // 8, pl.ds((k % 8) * L, L)] = jnp.zeros((L,), jnp.float32)
        return carry
    lax.fori_loop(0, ROWS_TILE * 8, _zm, 0)

    def _zero_mine():
        for t in range(ROWS_TILE // 4):
            pltpu.sync_copy(zbuf, acc.at[pl.ds(s * ROWS_TILE + t * 4, 4)])
        pltpu.sync_copy(zmb, macc.at[pl.ds(s * ROWS_TILE, ROWS_TILE)])

    _zero_mine()

    @pl.when(s == 0)
    def _zdump():
        pltpu.sync_copy(zbuf, acc.at[pl.ds(SEG, 4)])
        pltpu.sync_copy(zbuf, acc.at[pl.ds(SEG + 4, 4)])
        pltpu.sync_copy(zmb.at[pl.ds(0, 8)], macc.at[pl.ds(SEG, 8)])

    plsc.subcore_barrier()

    def _pass(j, carry):
        i = c * (NPASS // NC) + j
        seg0 = i * SEG
        vb = bnd_v[pl.ds(i, L)]
        n0 = vb[0]
        n1 = vb[1]
        base0 = (n0 // 8) * 8
        span = n1 - base0
        nwin = (span + WCAP - 1) // WCAP

        def _win(w, carry2):
            off = base0 + w * WCAP
            pltpu.sync_copy(lg_hbm.at[pl.ds(off, WCAP)], lgb)
            pltpu.sync_copy(lc_hbm.at[pl.ds(off, WCAP)], lcb)
            lim = jnp.minimum(WCAP, n1 - off)
            ngroups = (lim + GROUP - 1) // GROUP
            K = (ngroups - s + NS - 1) // NS

            def _make_idx(k, slot):
                g = s + k * NS
                vlg = lgb[pl.ds(g * GROUP, L)]
                vlc = lcb[pl.ds(g * GROUP, L)]
                loc = vlc - seg0
                ok = (loc >= 0) & (loc < SEG)
                gidx[slot, pl.ds(0, L)] = vlg
                lcx[slot, pl.ds(0, L)] = jnp.where(ok, loc, SEG)

            def _start(slot, gb, mb, sv, sm):
                pltpu.make_async_copy(th_hbm.at[gidx.at[slot]], gb, sv).start()
                pltpu.make_async_copy(mg_hbm.at[gidx.at[slot]], mb, sm).start()

            def _drain(slot, gb, mb, sv, sm):
                pltpu.make_async_copy(th_hbm.at[gidx.at[slot]], gb, sv).wait()
                pltpu.sync_copy(gb, acc.at[lcx.at[slot]], add=True)
                pltpu.make_async_copy(mg_hbm.at[gidx.at[slot]], mb, sm).wait()
                pltpu.sync_copy(mb, macc.at[lcx.at[slot]], add=True)

            @pl.when(K > 0)
            def _pro():
                _make_idx(0, 0)
                _start(0, gbuf0, mbuf0, semv0, semm0)

            def _body(m, carry3):
                b = 2 * m + 1

                @pl.when(b < K)
                def _s1():
                    _make_idx(b, 1)
                    _start(1, gbuf1, mbuf1, semv1, semm1)

                _drain(0, gbuf0, mbuf0, semv0, semm0)

                @pl.when(2 * m + 2 < K)
                def _s0():
                    _make_idx(2 * m + 2, 0)
                    _start(0, gbuf0, mbuf0, semv0, semm0)

                @pl.when(b < K)
                def _d1():
                    _drain(1, gbuf1, mbuf1, semv1, semm1)
                return carry3
            lax.fori_loop(0, (K + 1) // 2, _body, 0)
            return carry2
        lax.fori_loop(0, nwin, _win, 0)

        plsc.subcore_barrier()

        pltpu.sync_copy(acc.at[pl.ds(s * ROWS_TILE, ROWS_TILE)],
                        agg_hbm.at[pl.ds(seg0 + s * ROWS_TILE, ROWS_TILE)])
        pltpu.sync_copy(macc.at[pl.ds(s * ROWS_TILE, ROWS_TILE)],
                        magg_hbm.at[pl.ds(seg0 + s * ROWS_TILE, ROWS_TILE)])
        _zero_mine()

        @pl.when(s == 0)
        def _zdump2():
            pltpu.sync_copy(zbuf, acc.at[pl.ds(SEG, 4)])
            pltpu.sync_copy(zbuf, acc.at[pl.ds(SEG + 4, 4)])
            pltpu.sync_copy(zmb.at[pl.ds(0, 8)], macc.at[pl.ds(SEG, 8)])

        plsc.subcore_barrier()
        return carry
    lax.fori_loop(0, NPASS // NC, _pass, 0)


def _segment_aggregate(th3, maskg, lg1, lc1, bounds):
    f = pl.kernel(
        _seg_body,
        out_type=[
            jax.ShapeDtypeStruct((B, P, H), jnp.float32),
            jax.ShapeDtypeStruct((B, H), jnp.float32),
        ],
        mesh=plsc.VectorSubcoreMesh(core_axis_name="c", subcore_axis_name="s"),
        scratch_types=[
            pltpu.VMEM((WCAP,), jnp.int32),
            pltpu.VMEM((WCAP,), jnp.int32),
            pltpu.VMEM((2, GROUP), jnp.int32),
            pltpu.VMEM((2, GROUP), jnp.int32),
            pltpu.VMEM((GROUP, P, H), jnp.float32),
            pltpu.VMEM((GROUP, P, H), jnp.float32),
            pltpu.VMEM((GROUP, H), jnp.float32),
            pltpu.VMEM((GROUP, H), jnp.float32),
            pltpu.VMEM((4, P, H), jnp.float32),
            pltpu.VMEM((ROWS_TILE, H), jnp.float32),
            pltpu.VMEM((NBOUND,), jnp.int32),
            pltpu.VMEM_SHARED((SEG + 8, P, H), jnp.float32),
            pltpu.VMEM_SHARED((SEG + 8, H), jnp.float32),
            pltpu.SemaphoreType.DMA,
            pltpu.SemaphoreType.DMA,
            pltpu.SemaphoreType.DMA,
            pltpu.SemaphoreType.DMA,
        ],
    )
    return f(th3, maskg, lg1, lc1, bounds)


def _pool_mlp_body(ag_ref, mg_ref, wp_ref, wi1_ref, bi1_ref, wi2_ref,
                   bi2_ref, out_ref):
    wp = wp_ref[...]
    m = jnp.max(wp, axis=1, keepdims=True)
    e = jnp.exp(wp - m)
    a = e / jnp.sum(e, axis=1, keepdims=True)
    mg = mg_ref[...][:, :P]
    wm = (mg > 0).astype(jnp.float32) * a
    y = ag_ref[...] * wm[:, :, None]
    pooled = jnp.sum(y, axis=1)
    h1 = jnp.dot(pooled, wi1_ref[...], preferred_element_type=jnp.float32)
    h1 = h1 + bi1_ref[...]
    h1 = jnp.where(h1 >= 0, h1, 0.01 * h1)
    o = jnp.dot(h1, wi2_ref[...], preferred_element_type=jnp.float32)
    out_ref[...] = o + bi2_ref[...]


def _pool_mlp(agg3, magg, wpr, Wi1, bi1r, Wi2, bi2r):
    TB = 256
    grid = B // TB
    return pl.pallas_call(
        _pool_mlp_body,
        grid=(grid,),
        in_specs=[
            pl.BlockSpec((TB, P, H), lambda i: (i, 0, 0)),
            pl.BlockSpec((TB, H), lambda i: (i, 0)),
            pl.BlockSpec((1, P), lambda i: (0, 0)),
            pl.BlockSpec((H, H), lambda i: (0, 0)),
            pl.BlockSpec((1, H), lambda i: (0, 0)),
            pl.BlockSpec((H, PCA), lambda i: (0, 0)),
            pl.BlockSpec((1, PCA), lambda i: (0, 0)),
        ],
        out_specs=pl.BlockSpec((TB, PCA), lambda i: (i, 0)),
        out_shape=jax.ShapeDtypeStruct((B, PCA), jnp.float32),
    )(agg3, magg, wpr, Wi1, bi1r, Wi2, bi2r)


def kernel(table, W1, b1, w_path, Wi1, bi1, Wi2, bi2, locs_gene, locs_combos):
    tt = jnp.swapaxes(table, 1, 2)
    th3, maskg = _table_transform(tt, W1, b1.reshape(1, H))
    lg1 = jnp.concatenate([locs_gene, jnp.zeros((WCAP,), jnp.int32)])
    lc1 = jnp.concatenate([locs_combos, jnp.full((WCAP,), B, jnp.int32)])
    bounds = jnp.searchsorted(locs_combos,
                              jnp.arange(0, B + 1, SEG)).astype(jnp.int32)
    bounds = jnp.concatenate(
        [bounds, jnp.full((NBOUND - NPASS - 1,), N, jnp.int32)])
    agg3, magg = _segment_aggregate(th3, maskg, lg1, lc1, bounds)
    out = _pool_mlp(agg3, magg, w_path.reshape(1, P), Wi1,
                    bi1.reshape(1, H), Wi2, bi2.reshape(1, PCA))
    return (out, jnp.swapaxes(agg3, 1, 2))


# speedup vs baseline: 1.0952x; 1.0952x over previous
import functools

import jax
import jax.numpy as jnp
import numpy as np
from jax import lax
from jax.experimental import pallas as pl
from jax.experimental.pallas import tpu as pltpu
from jax.experimental.pallas import tpu_sc as plsc

G, D, P = 20000, 128, 16
H = 128
PCA = 512
B = 16384
N = 32768

HP = H * P
NC, NS, L = 2, 16, 16
B2 = B // NC
SEG = 512
NPASS = B // SEG
WCAP = 2048
GROUP = 16
ROWS_TILE = SEG // NS
NBOUND = 48


def _table_transform_body(x_ref, w_ref, b_ref, th_ref, m_ref):
    x = x_ref[...]
    tg = x.shape[0]
    x2 = x.reshape(tg * P, D)
    y = jnp.dot(x2, w_ref[...], preferred_element_type=jnp.float32)
    y = y + b_ref[...]
    y = jnp.where(y >= 0, y, 0.01 * y)
    th_ref[...] = y.reshape(tg, P, H)
    s = jnp.sum(x, axis=2)
    m = (s != 0).astype(jnp.float32)
    m_ref[...] = jnp.concatenate(
        [m, jnp.zeros((tg, H - P), jnp.float32)], axis=1)


def _table_transform(tt, W1, b1r):
    TG = 400
    grid = G // TG
    return pl.pallas_call(
        _table_transform_body,
        grid=(grid,),
        in_specs=[
            pl.BlockSpec((TG, P, D), lambda i: (i, 0, 0)),
            pl.BlockSpec((D, H), lambda i: (0, 0)),
            pl.BlockSpec((1, H), lambda i: (0, 0)),
        ],
        out_specs=[
            pl.BlockSpec((TG, P, H), lambda i: (i, 0, 0)),
            pl.BlockSpec((TG, H), lambda i: (i, 0)),
        ],
        out_shape=[
            jax.ShapeDtypeStruct((G, P, H), jnp.float32),
            jax.ShapeDtypeStruct((G, H), jnp.float32),
        ],
    )(tt, W1, b1r)


def _seg_body(th_hbm, mg_hbm, lg_hbm, lc_hbm, bnd_hbm, agg_hbm, magg_hbm,
              lgb, lcb, gidx, lcx, gbuf0, gbuf1, mbuf0, mbuf1, zbuf, zmb,
              bnd_v, acc, macc, semv0, semv1, semm0, semm1):
    c = lax.axis_index("c")
    s = lax.axis_index("s")

    pltpu.sync_copy(bnd_hbm, bnd_v)

    def _zb(k, carry):
        zbuf[k // (P * 8), (k // 8) % P, pl.ds((k % 8) * L, L)] = (
            jnp.zeros((L,), jnp.float32))
        return carry
    lax.fori_loop(0, 4 * P * 8, _zb, 0)

    def _zm(k, carry):
        zmb[k // 8, pl.ds((k % 8) * L, L)] = jnp.zeros((L,), jnp.float32)
        return carry
    lax.fori_loop(0, ROWS_TILE * 8, _zm, 0)

    def _zero_mine():
        for t in range(ROWS_TILE // 4):
            pltpu.sync_copy(zbuf, acc.at[pl.ds(s * ROWS_TILE + t * 4, 4)])
        pltpu.sync_copy(zmb, macc.at[pl.ds(s * ROWS_TILE, ROWS_TILE)])

    _zero_mine()

    @pl.when(s == 0)
    def _zdump():
        pltpu.sync_copy(zbuf, acc.at[pl.ds(SEG, 4)])
        pltpu.sync_copy(zbuf, acc.at[pl.ds(SEG + 4, 4)])
        pltpu.sync_copy(zmb.at[pl.ds(0, 8)], macc.at[pl.ds(SEG, 8)])

    plsc.subcore_barrier()

    def _pass(j, carry):
        i = c * (NPASS // NC) + j
        seg0 = i * SEG
        vb = bnd_v[pl.ds(i, L)]
        n0 = vb[0]
        n1 = vb[1]
        base0 = (n0 // 8) * 8
        span = n1 - base0
        nwin = (span + WCAP - 1) // WCAP

        def _win(w, carry2):
            off = base0 + w * WCAP
            pltpu.sync_copy(lg_hbm.at[pl.ds(off, WCAP)], lgb)
            pltpu.sync_copy(lc_hbm.at[pl.ds(off, WCAP)], lcb)
            lim = jnp.minimum(WCAP, n1 - off)
            ngroups = (lim + GROUP - 1) // GROUP
            K = (ngroups - s + NS - 1) // NS

            def _make_idx(k, slot):
                g = s + k * NS
                vlg = lgb[pl.ds(g * GROUP, L)]
                vlc = lcb[pl.ds(g * GROUP, L)]
                loc = vlc - seg0
                ok = (loc >= 0) & (loc < SEG)
                gidx[slot, pl.ds(0, L)] = vlg
                lcx[slot, pl.ds(0, L)] = jnp.where(ok, loc, SEG)

            def _start(slot, gb, mb, sv, sm):
                pltpu.make_async_copy(th_hbm.at[gidx.at[slot]], gb, sv).start()
                pltpu.make_async_copy(mg_hbm.at[gidx.at[slot]], mb, sm).start()

            def _drain(slot, gb, mb, sv, sm):
                pltpu.make_async_copy(th_hbm.at[gidx.at[slot]], gb, sv).wait()
                pltpu.sync_copy(gb, acc.at[lcx.at[slot]], add=True)
                pltpu.make_async_copy(mg_hbm.at[gidx.at[slot]], mb, sm).wait()
                pltpu.sync_copy(mb, macc.at[lcx.at[slot]], add=True)

            def _body(m, carry3):
                _make_idx(m, 0)
                _start(0, gbuf0, mbuf0, semv0, semm0)
                _drain(0, gbuf0, mbuf0, semv0, semm0)
                return carry3
            lax.fori_loop(0, K, _body, 0)
            return carry2
        lax.fori_loop(0, nwin, _win, 0)

        plsc.subcore_barrier()

        pltpu.sync_copy(acc.at[pl.ds(s * ROWS_TILE, ROWS_TILE)],
                        agg_hbm.at[pl.ds(seg0 + s * ROWS_TILE, ROWS_TILE)])
        pltpu.sync_copy(macc.at[pl.ds(s * ROWS_TILE, ROWS_TILE)],
                        magg_hbm.at[pl.ds(seg0 + s * ROWS_TILE, ROWS_TILE)])
        _zero_mine()

        @pl.when(s == 0)
        def _zdump2():
            pltpu.sync_copy(zbuf, acc.at[pl.ds(SEG, 4)])
            pltpu.sync_copy(zbuf, acc.at[pl.ds(SEG + 4, 4)])
            pltpu.sync_copy(zmb.at[pl.ds(0, 8)], macc.at[pl.ds(SEG, 8)])

        plsc.subcore_barrier()
        return carry
    lax.fori_loop(0, NPASS // NC, _pass, 0)


def _segment_aggregate(th3, maskg, lg1, lc1, bounds):
    f = pl.kernel(
        _seg_body,
        out_type=[
            jax.ShapeDtypeStruct((B, P, H), jnp.float32),
            jax.ShapeDtypeStruct((B, H), jnp.float32),
        ],
        mesh=plsc.VectorSubcoreMesh(core_axis_name="c", subcore_axis_name="s"),
        scratch_types=[
            pltpu.VMEM((WCAP,), jnp.int32),
            pltpu.VMEM((WCAP,), jnp.int32),
            pltpu.VMEM((2, GROUP), jnp.int32),
            pltpu.VMEM((2, GROUP), jnp.int32),
            pltpu.VMEM((GROUP, P, H), jnp.float32),
            pltpu.VMEM((1, P, H), jnp.float32),
            pltpu.VMEM((GROUP, H), jnp.float32),
            pltpu.VMEM((1, H), jnp.float32),
            pltpu.VMEM((4, P, H), jnp.float32),
            pltpu.VMEM((ROWS_TILE, H), jnp.float32),
            pltpu.VMEM((NBOUND,), jnp.int32),
            pltpu.VMEM_SHARED((SEG + 8, P, H), jnp.float32),
            pltpu.VMEM_SHARED((SEG + 8, H), jnp.float32),
            pltpu.SemaphoreType.DMA,
            pltpu.SemaphoreType.DMA,
            pltpu.SemaphoreType.DMA,
            pltpu.SemaphoreType.DMA,
        ],
    )
    return f(th3, maskg, lg1, lc1, bounds)


def _pool_mlp_body(ag_ref, mg_ref, wp_ref, wi1_ref, bi1_ref, wi2_ref,
                   bi2_ref, out_ref):
    wp = wp_ref[...]
    m = jnp.max(wp, axis=1, keepdims=True)
    e = jnp.exp(wp - m)
    a = e / jnp.sum(e, axis=1, keepdims=True)
    mg = mg_ref[...][:, :P]
    wm = (mg > 0).astype(jnp.float32) * a
    y = ag_ref[...] * wm[:, :, None]
    pooled = jnp.sum(y, axis=1)
    h1 = jnp.dot(pooled, wi1_ref[...], preferred_element_type=jnp.float32)
    h1 = h1 + bi1_ref[...]
    h1 = jnp.where(h1 >= 0, h1, 0.01 * h1)
    o = jnp.dot(h1, wi2_ref[...], preferred_element_type=jnp.float32)
    out_ref[...] = o + bi2_ref[...]


def _pool_mlp(agg3, magg, wpr, Wi1, bi1r, Wi2, bi2r):
    TB = 256
    grid = B // TB
    return pl.pallas_call(
        _pool_mlp_body,
        grid=(grid,),
        in_specs=[
            pl.BlockSpec((TB, P, H), lambda i: (i, 0, 0)),
            pl.BlockSpec((TB, H), lambda i: (i, 0)),
            pl.BlockSpec((1, P), lambda i: (0, 0)),
            pl.BlockSpec((H, H), lambda i: (0, 0)),
            pl.BlockSpec((1, H), lambda i: (0, 0)),
            pl.BlockSpec((H, PCA), lambda i: (0, 0)),
            pl.BlockSpec((1, PCA), lambda i: (0, 0)),
        ],
        out_specs=pl.BlockSpec((TB, PCA), lambda i: (i, 0)),
        out_shape=jax.ShapeDtypeStruct((B, PCA), jnp.float32),
    )(agg3, magg, wpr, Wi1, bi1r, Wi2, bi2r)


def kernel(table, W1, b1, w_path, Wi1, bi1, Wi2, bi2, locs_gene, locs_combos):
    tt = jnp.swapaxes(table, 1, 2)
    th3, maskg = _table_transform(tt, W1, b1.reshape(1, H))
    lg1 = jnp.concatenate([locs_gene, jnp.zeros((WCAP,), jnp.int32)])
    lc1 = jnp.concatenate([locs_combos, jnp.full((WCAP,), B, jnp.int32)])
    bounds = jnp.searchsorted(locs_combos,
                              jnp.arange(0, B + 1, SEG)).astype(jnp.int32)
    bounds = jnp.concatenate(
        [bounds, jnp.full((NBOUND - NPASS - 1,), N, jnp.int32)])
    agg3, magg = _segment_aggregate(th3, maskg, lg1, lc1, bounds)
    out = _pool_mlp(agg3, magg, w_path.reshape(1, P), Wi1,
                    bi1.reshape(1, H), Wi2, bi2.reshape(1, PCA))
    return (out, jnp.swapaxes(agg3, 1, 2))


# speedup vs baseline: 1.1196x; 1.0223x over previous
import functools

import jax
import jax.numpy as jnp
import numpy as np
from jax import lax
from jax.experimental import pallas as pl
from jax.experimental.pallas import tpu as pltpu
from jax.experimental.pallas import tpu_sc as plsc

G, D, P = 20000, 128, 16
H = 128
PCA = 512
B = 16384
N = 32768

HP = H * P
NC, NS, L = 2, 16, 16
B2 = B // NC
SEG = 512
NPASS = B // SEG
WCAP = 2048
GROUP = 16
ROWS_TILE = SEG // NS
NBOUND = 48


def _table_transform_body(x_ref, w_ref, b_ref, th_ref, m_ref):
    x = x_ref[...]
    tg = x.shape[0]
    x2 = x.reshape(tg * P, D)
    y = jnp.dot(x2, w_ref[...], preferred_element_type=jnp.float32)
    y = y + b_ref[...]
    y = jnp.where(y >= 0, y, 0.01 * y)
    th_ref[...] = y.reshape(tg, P, H)
    s = jnp.sum(x, axis=2)
    m = (s != 0).astype(jnp.float32)
    m_ref[...] = jnp.concatenate(
        [m, jnp.zeros((tg, H - P), jnp.float32)], axis=1)


def _table_transform(tt, W1, b1r):
    TG = 400
    grid = G // TG
    return pl.pallas_call(
        _table_transform_body,
        grid=(grid,),
        in_specs=[
            pl.BlockSpec((TG, P, D), lambda i: (i, 0, 0)),
            pl.BlockSpec((D, H), lambda i: (0, 0)),
            pl.BlockSpec((1, H), lambda i: (0, 0)),
        ],
        out_specs=[
            pl.BlockSpec((TG, P, H), lambda i: (i, 0, 0)),
            pl.BlockSpec((TG, H), lambda i: (i, 0)),
        ],
        out_shape=[
            jax.ShapeDtypeStruct((G, P, H), jnp.float32),
            jax.ShapeDtypeStruct((G, H), jnp.float32),
        ],
    )(tt, W1, b1r)


def _seg_body(th_hbm, mg_hbm, lg_hbm, lc_hbm, bnd_hbm, agg_hbm, magg_hbm,
              lgb, lcb, gidx, lcx, gbuf0, gbuf1, mbuf0, mbuf1, zbuf, zmb,
              bnd_v, acc, macc, semv0, semv1, semm0, semm1):
    c = lax.axis_index("c")
    s = lax.axis_index("s")

    pltpu.sync_copy(bnd_hbm, bnd_v)

    def _zb(k, carry):
        zbuf[k // (P * 8), (k // 8) % P, pl.ds((k % 8) * L, L)] = (
            jnp.zeros((L,), jnp.float32))
        return carry
    lax.fori_loop(0, 4 * P * 8, _zb, 0)

    def _zm(k, carry):
        zmb[k // 8, pl.ds((k % 8) * L, L)] = jnp.zeros((L,), jnp.float32)
        return carry
    lax.fori_loop(0, ROWS_TILE * 8, _zm, 0)

    def _zero_mine():
        for t in range(ROWS_TILE // 4):
            pltpu.sync_copy(zbuf, acc.at[pl.ds(s * ROWS_TILE + t * 4, 4)])
        pltpu.sync_copy(zmb, macc.at[pl.ds(s * ROWS_TILE, ROWS_TILE)])

    _zero_mine()

    @pl.when(s == 0)
    def _zdump():
        pltpu.sync_copy(zbuf, acc.at[pl.ds(SEG, 4)])
        pltpu.sync_copy(zbuf, acc.at[pl.ds(SEG + 4, 4)])
        pltpu.sync_copy(zmb.at[pl.ds(0, 8)], macc.at[pl.ds(SEG, 8)])

    plsc.subcore_barrier()

    def _pass(j, carry):
        i = c * (NPASS // NC) + j
        seg0 = i * SEG
        vb = bnd_v[pl.ds(i, L)]
        n0 = vb[0]
        n1 = vb[1]
        base0 = (n0 // 8) * 8
        span = n1 - base0
        nwin = (span + WCAP - 1) // WCAP

        def _win(w, carry2):
            off = base0 + w * WCAP
            pltpu.make_async_copy(lg_hbm.at[pl.ds(off, WCAP)], lgb, semv1).start()
            pltpu.make_async_copy(lc_hbm.at[pl.ds(off, WCAP)], lcb, semm1).start()
            pltpu.make_async_copy(lg_hbm.at[pl.ds(off, WCAP)], lgb, semv1).wait()
            pltpu.make_async_copy(lc_hbm.at[pl.ds(off, WCAP)], lcb, semm1).wait()
            lim = jnp.minimum(WCAP, n1 - off)
            ngroups = (lim + GROUP - 1) // GROUP
            K = (ngroups - s + NS - 1) // NS

            def _make_idx(k, slot):
                g = s + k * NS
                vlg = lgb[pl.ds(g * GROUP, L)]
                vlc = lcb[pl.ds(g * GROUP, L)]
                loc = vlc - seg0
                ok = (loc >= 0) & (loc < SEG)
                gidx[slot, pl.ds(0, L)] = vlg
                lcx[slot, pl.ds(0, L)] = jnp.where(ok, loc, SEG)

            def _start(slot, gb, mb, sv, sm):
                pltpu.make_async_copy(th_hbm.at[gidx.at[slot]], gb, sv).start()
                pltpu.make_async_copy(mg_hbm.at[gidx.at[slot]], mb, sm).start()

            def _drain(slot, gb, mb, sv, sm):
                pltpu.make_async_copy(th_hbm.at[gidx.at[slot]], gb, sv).wait()
                pltpu.sync_copy(gb, acc.at[lcx.at[slot]], add=True)
                pltpu.make_async_copy(mg_hbm.at[gidx.at[slot]], mb, sm).wait()
                pltpu.sync_copy(mb, macc.at[lcx.at[slot]], add=True)

            def _body(m, carry3):
                _make_idx(m, 0)
                _start(0, gbuf0, mbuf0, semv0, semm0)
                _drain(0, gbuf0, mbuf0, semv0, semm0)
                return carry3
            lax.fori_loop(0, K, _body, 0)
            return carry2
        lax.fori_loop(0, nwin, _win, 0)

        plsc.subcore_barrier()

        pltpu.sync_copy(acc.at[pl.ds(s * ROWS_TILE, ROWS_TILE)],
                        agg_hbm.at[pl.ds(seg0 + s * ROWS_TILE, ROWS_TILE)])
        pltpu.sync_copy(macc.at[pl.ds(s * ROWS_TILE, ROWS_TILE)],
                        magg_hbm.at[pl.ds(seg0 + s * ROWS_TILE, ROWS_TILE)])
        _zero_mine()

        @pl.when(s == 0)
        def _zdump2():
            pltpu.sync_copy(zbuf, acc.at[pl.ds(SEG, 4)])
            pltpu.sync_copy(zbuf, acc.at[pl.ds(SEG + 4, 4)])
            pltpu.sync_copy(zmb.at[pl.ds(0, 8)], macc.at[pl.ds(SEG, 8)])

        plsc.subcore_barrier()
        return carry
    lax.fori_loop(0, NPASS // NC, _pass, 0)


def _segment_aggregate(th3, maskg, lg1, lc1, bounds):
    f = pl.kernel(
        _seg_body,
        out_type=[
            jax.ShapeDtypeStruct((B, P, H), jnp.float32),
            jax.ShapeDtypeStruct((B, H), jnp.float32),
        ],
        mesh=plsc.VectorSubcoreMesh(core_axis_name="c", subcore_axis_name="s"),
        scratch_types=[
            pltpu.VMEM((WCAP,), jnp.int32),
            pltpu.VMEM((WCAP,), jnp.int32),
            pltpu.VMEM((2, GROUP), jnp.int32),
            pltpu.VMEM((2, GROUP), jnp.int32),
            pltpu.VMEM((GROUP, P, H), jnp.float32),
            pltpu.VMEM((1, P, H), jnp.float32),
            pltpu.VMEM((GROUP, H), jnp.float32),
            pltpu.VMEM((1, H), jnp.float32),
            pltpu.VMEM((4, P, H), jnp.float32),
            pltpu.VMEM((ROWS_TILE, H), jnp.float32),
            pltpu.VMEM((NBOUND,), jnp.int32),
            pltpu.VMEM_SHARED((SEG + 8, P, H), jnp.float32),
            pltpu.VMEM_SHARED((SEG + 8, H), jnp.float32),
            pltpu.SemaphoreType.DMA,
            pltpu.SemaphoreType.DMA,
            pltpu.SemaphoreType.DMA,
            pltpu.SemaphoreType.DMA,
        ],
    )
    return f(th3, maskg, lg1, lc1, bounds)


def _pool_mlp_body(ag_ref, mg_ref, wp_ref, wi1_ref, bi1_ref, wi2_ref,
                   bi2_ref, out_ref):
    wp = wp_ref[...]
    m = jnp.max(wp, axis=1, keepdims=True)
    e = jnp.exp(wp - m)
    a = e / jnp.sum(e, axis=1, keepdims=True)
    mg = mg_ref[...][:, :P]
    wm = (mg > 0).astype(jnp.float32) * a
    y = ag_ref[...] * wm[:, :, None]
    pooled = jnp.sum(y, axis=1)
    h1 = jnp.dot(pooled, wi1_ref[...], preferred_element_type=jnp.float32)
    h1 = h1 + bi1_ref[...]
    h1 = jnp.where(h1 >= 0, h1, 0.01 * h1)
    o = jnp.dot(h1, wi2_ref[...], preferred_element_type=jnp.float32)
    out_ref[...] = o + bi2_ref[...]


def _pool_mlp(agg3, magg, wpr, Wi1, bi1r, Wi2, bi2r):
    TB = 256
    grid = B // TB
    return pl.pallas_call(
        _pool_mlp_body,
        grid=(grid,),
        in_specs=[
            pl.BlockSpec((TB, P, H), lambda i: (i, 0, 0)),
            pl.BlockSpec((TB, H), lambda i: (i, 0)),
            pl.BlockSpec((1, P), lambda i: (0, 0)),
            pl.BlockSpec((H, H), lambda i: (0, 0)),
            pl.BlockSpec((1, H), lambda i: (0, 0)),
            pl.BlockSpec((H, PCA), lambda i: (0, 0)),
            pl.BlockSpec((1, PCA), lambda i: (0, 0)),
        ],
        out_specs=pl.BlockSpec((TB, PCA), lambda i: (i, 0)),
        out_shape=jax.ShapeDtypeStruct((B, PCA), jnp.float32),
    )(agg3, magg, wpr, Wi1, bi1r, Wi2, bi2r)


def kernel(table, W1, b1, w_path, Wi1, bi1, Wi2, bi2, locs_gene, locs_combos):
    tt = jnp.swapaxes(table, 1, 2)
    th3, maskg = _table_transform(tt, W1, b1.reshape(1, H))
    lg1 = jnp.concatenate([locs_gene, jnp.zeros((WCAP,), jnp.int32)])
    lc1 = jnp.concatenate([locs_combos, jnp.full((WCAP,), B, jnp.int32)])
    bounds = jnp.searchsorted(locs_combos,
                              jnp.arange(0, B + 1, SEG)).astype(jnp.int32)
    bounds = jnp.concatenate(
        [bounds, jnp.full((NBOUND - NPASS - 1,), N, jnp.int32)])
    agg3, magg = _segment_aggregate(th3, maskg, lg1, lc1, bounds)
    out = _pool_mlp(agg3, magg, w_path.reshape(1, P), Wi1,
                    bi1.reshape(1, H), Wi2, bi2.reshape(1, PCA))
    return (out, jnp.swapaxes(agg3, 1, 2))


# speedup vs baseline: 1.1740x; 1.0485x over previous
import functools

import jax
import jax.numpy as jnp
import numpy as np
from jax import lax
from jax.experimental import pallas as pl
from jax.experimental.pallas import tpu as pltpu
from jax.experimental.pallas import tpu_sc as plsc

G, D, P = 20000, 128, 16
H = 128
PCA = 512
B = 16384
N = 32768

HP = H * P
NC, NS, L = 2, 16, 16
B2 = B // NC
SEG = 512
NPASS = B // SEG
WCAP = 2048
GROUP = 16
ROWS_TILE = SEG // NS
NBOUND = 48


def _table_transform_body(x_ref, w_ref, b_ref, th_ref, m_ref):
    x = x_ref[...]
    tg = x.shape[0]
    x2 = x.reshape(tg * P, D)
    y = jnp.dot(x2, w_ref[...], preferred_element_type=jnp.float32)
    y = y + b_ref[...]
    y = jnp.where(y >= 0, y, 0.01 * y)
    th_ref[...] = y.reshape(tg, P, H)
    s = jnp.sum(x, axis=2)
    m = (s != 0).astype(jnp.float32)
    m_ref[...] = jnp.concatenate(
        [m, jnp.zeros((tg, H - P), jnp.float32)], axis=1)


def _table_transform(tt, W1, b1r):
    TG = 1000
    grid = G // TG
    return pl.pallas_call(
        _table_transform_body,
        grid=(grid,),
        in_specs=[
            pl.BlockSpec((TG, P, D), lambda i: (i, 0, 0)),
            pl.BlockSpec((D, H), lambda i: (0, 0)),
            pl.BlockSpec((1, H), lambda i: (0, 0)),
        ],
        out_specs=[
            pl.BlockSpec((TG, P, H), lambda i: (i, 0, 0)),
            pl.BlockSpec((TG, H), lambda i: (i, 0)),
        ],
        out_shape=[
            jax.ShapeDtypeStruct((G, P, H), jnp.float32),
            jax.ShapeDtypeStruct((G, H), jnp.float32),
        ],
    )(tt, W1, b1r)


def _seg_body(th_hbm, mg_hbm, lg_hbm, lc_hbm, bnd_hbm, agg_hbm, magg_hbm,
              lgb, lcb, gidx, lcx, gbuf0, gbuf1, mbuf0, mbuf1, zbuf, zmb,
              bnd_v, acc, macc, semv0, semv1, semm0, semm1):
    c = lax.axis_index("c")
    s = lax.axis_index("s")

    pltpu.sync_copy(bnd_hbm, bnd_v)

    def _zb(k, carry):
        zbuf[k // (P * 8), (k // 8) % P, pl.ds((k % 8) * L, L)] = (
            jnp.zeros((L,), jnp.float32))
        return carry
    lax.fori_loop(0, 4 * P * 8, _zb, 0)

    def _zm(k, carry):
        zmb[k // 8, pl.ds((k % 8) * L, L)] = jnp.zeros((L,), jnp.float32)
        return carry
    lax.fori_loop(0, ROWS_TILE * 8, _zm, 0)

    def _zero_mine():
        for t in range(ROWS_TILE // 4):
            pltpu.sync_copy(zbuf, acc.at[pl.ds(s * ROWS_TILE + t * 4, 4)])
        pltpu.sync_copy(zmb, macc.at[pl.ds(s * ROWS_TILE, ROWS_TILE)])

    _zero_mine()

    @pl.when(s == 0)
    def _zdump():
        pltpu.sync_copy(zbuf, acc.at[pl.ds(SEG, 4)])
        pltpu.sync_copy(zbuf, acc.at[pl.ds(SEG + 4, 4)])
        pltpu.sync_copy(zmb.at[pl.ds(0, 8)], macc.at[pl.ds(SEG, 8)])

    plsc.subcore_barrier()

    def _pass(j, carry):
        i = c * (NPASS // NC) + j
        seg0 = i * SEG
        vb = bnd_v[pl.ds(i, L)]
        n0 = vb[0]
        n1 = vb[1]
        base0 = (n0 // 8) * 8
        span = n1 - base0
        nwin = (span + WCAP - 1) // WCAP

        def _win(w, carry2):
            off = base0 + w * WCAP
            pltpu.make_async_copy(lg_hbm.at[pl.ds(off, WCAP)], lgb, semv1).start()
            pltpu.make_async_copy(lc_hbm.at[pl.ds(off, WCAP)], lcb, semm1).start()
            pltpu.make_async_copy(lg_hbm.at[pl.ds(off, WCAP)], lgb, semv1).wait()
            pltpu.make_async_copy(lc_hbm.at[pl.ds(off, WCAP)], lcb, semm1).wait()
            lim = jnp.minimum(WCAP, n1 - off)
            ngroups = (lim + GROUP - 1) // GROUP
            K = (ngroups - s + NS - 1) // NS

            def _make_idx(k, slot):
                g = s + k * NS
                vlg = lgb[pl.ds(g * GROUP, L)]
                vlc = lcb[pl.ds(g * GROUP, L)]
                loc = vlc - seg0
                ok = (loc >= 0) & (loc < SEG)
                gidx[slot, pl.ds(0, L)] = vlg
                lcx[slot, pl.ds(0, L)] = jnp.where(ok, loc, SEG)

            def _start(slot, gb, mb, sv, sm):
                pltpu.make_async_copy(th_hbm.at[gidx.at[slot]], gb, sv).start()
                pltpu.make_async_copy(mg_hbm.at[gidx.at[slot]], mb, sm).start()

            def _drain(slot, gb, mb, sv, sm):
                pltpu.make_async_copy(th_hbm.at[gidx.at[slot]], gb, sv).wait()
                pltpu.sync_copy(gb, acc.at[lcx.at[slot]], add=True)
                pltpu.make_async_copy(mg_hbm.at[gidx.at[slot]], mb, sm).wait()
                pltpu.sync_copy(mb, macc.at[lcx.at[slot]], add=True)

            def _body(m, carry3):
                _make_idx(m, 0)
                _start(0, gbuf0, mbuf0, semv0, semm0)
                _drain(0, gbuf0, mbuf0, semv0, semm0)
                return carry3
            lax.fori_loop(0, K, _body, 0)
            return carry2
        lax.fori_loop(0, nwin, _win, 0)

        plsc.subcore_barrier()

        pltpu.sync_copy(acc.at[pl.ds(s * ROWS_TILE, ROWS_TILE)],
                        agg_hbm.at[pl.ds(seg0 + s * ROWS_TILE, ROWS_TILE)])
        pltpu.sync_copy(macc.at[pl.ds(s * ROWS_TILE, ROWS_TILE)],
                        magg_hbm.at[pl.ds(seg0 + s * ROWS_TILE, ROWS_TILE)])
        _zero_mine()

        @pl.when(s == 0)
        def _zdump2():
            pltpu.sync_copy(zbuf, acc.at[pl.ds(SEG, 4)])
            pltpu.sync_copy(zbuf, acc.at[pl.ds(SEG + 4, 4)])
            pltpu.sync_copy(zmb.at[pl.ds(0, 8)], macc.at[pl.ds(SEG, 8)])

        plsc.subcore_barrier()
        return carry
    lax.fori_loop(0, NPASS // NC, _pass, 0)


def _segment_aggregate(th3, maskg, lg1, lc1, bounds):
    f = pl.kernel(
        _seg_body,
        out_type=[
            jax.ShapeDtypeStruct((B, P, H), jnp.float32),
            jax.ShapeDtypeStruct((B, H), jnp.float32),
        ],
        mesh=plsc.VectorSubcoreMesh(core_axis_name="c", subcore_axis_name="s"),
        scratch_types=[
            pltpu.VMEM((WCAP,), jnp.int32),
            pltpu.VMEM((WCAP,), jnp.int32),
            pltpu.VMEM((2, GROUP), jnp.int32),
            pltpu.VMEM((2, GROUP), jnp.int32),
            pltpu.VMEM((GROUP, P, H), jnp.float32),
            pltpu.VMEM((1, P, H), jnp.float32),
            pltpu.VMEM((GROUP, H), jnp.float32),
            pltpu.VMEM((1, H), jnp.float32),
            pltpu.VMEM((4, P, H), jnp.float32),
            pltpu.VMEM((ROWS_TILE, H), jnp.float32),
            pltpu.VMEM((NBOUND,), jnp.int32),
            pltpu.VMEM_SHARED((SEG + 8, P, H), jnp.float32),
            pltpu.VMEM_SHARED((SEG + 8, H), jnp.float32),
            pltpu.SemaphoreType.DMA,
            pltpu.SemaphoreType.DMA,
            pltpu.SemaphoreType.DMA,
            pltpu.SemaphoreType.DMA,
        ],
    )
    return f(th3, maskg, lg1, lc1, bounds)


def _pool_mlp_body(ag_ref, mg_ref, wp_ref, wi1_ref, bi1_ref, wi2_ref,
                   bi2_ref, out_ref):
    wp = wp_ref[...]
    m = jnp.max(wp, axis=1, keepdims=True)
    e = jnp.exp(wp - m)
    a = e / jnp.sum(e, axis=1, keepdims=True)
    mg = mg_ref[...][:, :P]
    wm = (mg > 0).astype(jnp.float32) * a
    y = ag_ref[...] * wm[:, :, None]
    pooled = jnp.sum(y, axis=1)
    h1 = jnp.dot(pooled, wi1_ref[...], preferred_element_type=jnp.float32)
    h1 = h1 + bi1_ref[...]
    h1 = jnp.where(h1 >= 0, h1, 0.01 * h1)
    o = jnp.dot(h1, wi2_ref[...], preferred_element_type=jnp.float32)
    out_ref[...] = o + bi2_ref[...]


def _pool_mlp(agg3, magg, wpr, Wi1, bi1r, Wi2, bi2r):
    TB = 512
    grid = B // TB
    return pl.pallas_call(
        _pool_mlp_body,
        grid=(grid,),
        in_specs=[
            pl.BlockSpec((TB, P, H), lambda i: (i, 0, 0)),
            pl.BlockSpec((TB, H), lambda i: (i, 0)),
            pl.BlockSpec((1, P), lambda i: (0, 0)),
            pl.BlockSpec((H, H), lambda i: (0, 0)),
            pl.BlockSpec((1, H), lambda i: (0, 0)),
            pl.BlockSpec((H, PCA), lambda i: (0, 0)),
            pl.BlockSpec((1, PCA), lambda i: (0, 0)),
        ],
        out_specs=pl.BlockSpec((TB, PCA), lambda i: (i, 0)),
        out_shape=jax.ShapeDtypeStruct((B, PCA), jnp.float32),
    )(agg3, magg, wpr, Wi1, bi1r, Wi2, bi2r)


def kernel(table, W1, b1, w_path, Wi1, bi1, Wi2, bi2, locs_gene, locs_combos):
    tt = jnp.swapaxes(table, 1, 2)
    th3, maskg = _table_transform(tt, W1, b1.reshape(1, H))
    lg1 = jnp.concatenate([locs_gene, jnp.zeros((WCAP,), jnp.int32)])
    lc1 = jnp.concatenate([locs_combos, jnp.full((WCAP,), B, jnp.int32)])
    bounds = jnp.searchsorted(locs_combos,
                              jnp.arange(0, B + 1, SEG)).astype(jnp.int32)
    bounds = jnp.concatenate(
        [bounds, jnp.full((NBOUND - NPASS - 1,), N, jnp.int32)])
    agg3, magg = _segment_aggregate(th3, maskg, lg1, lc1, bounds)
    out = _pool_mlp(agg3, magg, w_path.reshape(1, P), Wi1,
                    bi1.reshape(1, H), Wi2, bi2.reshape(1, PCA))
    return (out, jnp.swapaxes(agg3, 1, 2))
